# baseline (speedup 1.0000x reference)
import jax
import jax.numpy as jnp
from jax.experimental import pallas as pl

N = 10000
B = 32
H = 8
C = 56
HC = H * C


def _gat_layer(x, src, dst, ea, Wl, Wr, We, att, bias):
    n = x.shape[0]
    loop = jnp.arange(n)
    src2 = jnp.concatenate([src, loop])
    dst2 = jnp.concatenate([dst, loop])
    ea_loop = jnp.broadcast_to(ea.mean(axis=0, keepdims=True), (n, ea.shape[1]))
    ea2 = jnp.concatenate([ea, ea_loop], axis=0)
    xl = (x @ Wl).reshape(n, H, C)
    xr = (x @ Wr).reshape(n, H, C)
    e = (ea2 @ We).reshape(-1, H, C)
    m = xl[src2] + xr[dst2] + e
    m = jax.nn.leaky_relu(m, negative_slope=0.2)
    alpha = (m * att[None, :, :]).sum(axis=-1)
    amax = jax.ops.segment_max(alpha, dst2, num_segments=n)
    alpha = jnp.exp(alpha - amax[dst2])
    denom = jax.ops.segment_sum(alpha, dst2, num_segments=n)
    alpha = alpha / (denom[dst2] + 1e-16)
    out = jax.ops.segment_sum(xl[src2] * alpha[:, :, None], dst2, num_segments=n)
    return out.reshape(n, HC) + bias


def _batchnorm(x, gamma, beta):
    mean = x.mean(axis=0, keepdims=True)
    var = jnp.mean((x - mean) ** 2, axis=0, keepdims=True)
    return (x - mean) / jnp.sqrt(var + 1e-5) * gamma + beta


def _final_kernel(g_ref, w_ref, b_ref, o_ref):
    logits = g_ref[...] @ w_ref[...] + b_ref[...]
    mx = jnp.max(logits, axis=1, keepdims=True)
    z = logits - mx
    lse = jnp.log(jnp.sum(jnp.exp(z), axis=1, keepdims=True))
    o_ref[...] = z - lse


def kernel(x, edge_index, edge_attr, batch, Wl1, Wr1, We1, att1, b1, Wl2, Wr2, We2, att2, b2, gamma, beta, Wfc, bfc):
    src = edge_index[0]
    dst = edge_index[1]
    h = _gat_layer(x, src, dst, edge_attr, Wl1, Wr1, We1, att1, b1)
    h = jax.nn.elu(h)
    h = _batchnorm(h, gamma, beta)
    h = _gat_layer(h, src, dst, edge_attr, Wl2, Wr2, We2, att2, b2)
    sums = jax.ops.segment_sum(h, batch, num_segments=B)
    cnt = jax.ops.segment_sum(jnp.ones((h.shape[0], 1), jnp.float32), batch, num_segments=B)
    g = sums / jnp.maximum(cnt, 1.0)
    g = _batchnorm(g, gamma, beta)
    out = pl.pallas_call(
        _final_kernel,
        out_shape=jax.ShapeDtypeStruct((B, Wfc.shape[1]), jnp.float32),
    )(g, Wfc, bfc)
    return out


# speedup vs baseline: 14.4125x; 14.4125x over previous
import jax
import jax.numpy as jnp
from jax import lax
from jax.experimental import pallas as pl
from jax.experimental.pallas import tpu as pltpu
from jax.experimental.pallas import tpu_sc as plsc

N = 10000
E = 320000
B = 32
DE = 5
H = 8
HP = 4
C = 56
CP = 64
HC = H * C
HCP = H * CP
NCLS = 18

NCORE = 2
NSUB = 16
NW = NCORE * NSUB

K = 400
EPW_A = E // NW
NCH_A = EPW_A // K
EPW_B = E // NSUB
NCH_B = EPW_B // K
NR = N * HP

_f32 = jnp.float32
_i32 = jnp.int32


def _attn_body(srce_ref, dste_ref, ea_ref, xlp_ref, xrp_ref, wep_ref, attp_ref,
               alpha_ref,
               src_v, dst_v, eav, idx_s, idx_d, xls, xrd, alph_a, alph_b,
               wev, attv, sem, sem2):
  cid = lax.axis_index("c")
  sid = lax.axis_index("s")
  wid = sid * NCORE + cid

  pltpu.sync_copy(wep_ref, wev)
  pltpu.sync_copy(attp_ref, attv)
  lane15 = lax.broadcasted_iota(_i32, (16,), 0) == 15

  def chunk_body(i, _):
    base = wid * EPW_A + i * K
    pltpu.sync_copy(srce_ref.at[pl.ds(base, K)], src_v)
    pltpu.sync_copy(dste_ref.at[pl.ds(base, K)], dst_v)
    pltpu.sync_copy(ea_ref.at[pl.ds(base * DE, K * DE)], eav)

    def pair_body(hp, _):
      @plsc.parallel_loop(0, K // 16, 1, unroll=2)
      def _mk_idx(g):
        sl = pl.ds(g * 16, 16)
        idx_s[sl] = src_v[sl] * HP + hp
        idx_d[sl] = dst_v[sl] * HP + hp

      cp1 = pltpu.async_copy(xlp_ref.at[idx_s], xls, sem)
      cp2 = pltpu.async_copy(xrp_ref.at[idx_d], xrd, sem2)
      cp1.wait()
      cp2.wait()

      wq = [[wev[j * H + hp * 2 + q // 4, pl.ds((q % 4) * 16, 16)]
             for q in range(8)] for j in range(DE)]
      aq = [attv[hp * 2 + q // 4, pl.ds((q % 4) * 16, 16)] for q in range(8)]

      @plsc.parallel_loop(0, K, 1, unroll=2)
      def _edge(e):
        esplat = jnp.full((16,), e, _i32)
        ebase = jnp.full((16,), e * DE, _i32)
        ea_j = [plsc.load_gather(eav, [ebase + j]) for j in range(DE)]
        acc0 = None
        acc1 = None
        for q in range(8):
          sl = pl.ds(q * 16, 16)
          m = xls[e, sl] + xrd[e, sl]
          for j in range(DE):
            m = m + ea_j[j] * wq[j][q]
          m = jnp.maximum(m, m * 0.2)
          t = m * aq[q]
          if q < 4:
            acc0 = t if acc0 is None else acc0 + t
          else:
            acc1 = t if acc1 is None else acc1 + t
        plsc.store_scatter(alph_a, [esplat], plsc.cumsum(acc0), mask=lane15)
        plsc.store_scatter(alph_b, [esplat], plsc.cumsum(acc1), mask=lane15)

      @plsc.parallel_loop(0, K // 16, 1, unroll=2)
      def _expo(g):
        sl = pl.ds(g * 16, 16)
        alph_a[sl] = jnp.exp(alph_a[sl])
        alph_b[sl] = jnp.exp(alph_b[sl])

      pltpu.sync_copy(alph_a, alpha_ref.at[pl.ds((hp * 2) * E + base, K)])
      pltpu.sync_copy(alph_b, alpha_ref.at[pl.ds((hp * 2 + 1) * E + base, K)])
      return 0

    lax.fori_loop(0, HP, pair_body, 0)
    return 0

  lax.fori_loop(0, NCH_A, chunk_body, 0)


def _aggr_body(srce_ref, dste_ref, alpha_ref, xlp_ref,
               acc_out_ref,
               src_v, dst_v, idx_s80, dst80, rows, alph_a, alph_b, acc_sh,
               sem):
  cid = lax.axis_index("c")
  sid = lax.axis_index("s")
  lane8 = lax.broadcasted_iota(_i32, (16,), 0) == 8
  ones16 = jnp.ones((16,), _f32)
  KG = 80

  for p in range(2):
    hp = cid * 2 + p

    @plsc.parallel_loop(0, KG, 1, unroll=4)
    def _zero_rows(e):
      for q in range(8):
        rows[e, pl.ds(q * 16, 16)] = jnp.zeros((16,), _f32)

    @pl.when(sid < 10)
    def _():
      r0 = sid * 1000
      for z in range(12):
        pltpu.sync_copy(rows, acc_sh.at[pl.ds(r0 + z * KG, KG), :])
      pltpu.sync_copy(rows.at[pl.ds(0, 40), :],
                      acc_sh.at[pl.ds(r0 + 12 * KG, 40), :])
    plsc.subcore_barrier()

    def chunk_body(i, _):
      base = sid * EPW_B + i * K
      pltpu.sync_copy(srce_ref.at[pl.ds(base, K)], src_v)
      pltpu.sync_copy(dste_ref.at[pl.ds(base, K)], dst_v)
      pltpu.sync_copy(alpha_ref.at[pl.ds((hp * 2) * E + base, K)], alph_a)
      pltpu.sync_copy(alpha_ref.at[pl.ds((hp * 2 + 1) * E + base, K)], alph_b)

      for g in range(K // KG):
        @plsc.parallel_loop(0, KG // 16, 1, unroll=1)
        def _mk_idx(t):
          slo = pl.ds(t * 16, 16)
          sli = pl.ds(g * KG + t * 16, 16)
          idx_s80[slo] = src_v[sli] * HP + hp
          dst80[slo] = dst_v[sli]

        pltpu.async_copy(xlp_ref.at[idx_s80], rows, sem).wait()

        @plsc.parallel_loop(0, KG, 1, unroll=2)
        def _scale(e):
          esplat = jnp.full((16,), g * KG + e, _i32)
          a0 = plsc.load_gather(alph_a, [esplat])
          a1 = plsc.load_gather(alph_b, [esplat])
          for q in range(8):
            sl = pl.ds(q * 16, 16)
            r = rows[e, sl]
            if q == 3 or q == 7:
              r = jnp.where(lane8, ones16, r)
            rows[e, sl] = r * (a0 if q < 4 else a1)

        pltpu.sync_copy(rows, acc_sh.at[dst80], add=True)
      return 0

    lax.fori_loop(0, NCH_B, chunk_body, 0)
    plsc.subcore_barrier()

    @pl.when(sid < 10)
    def _():
      r0 = sid * 1000
      pltpu.sync_copy(acc_sh.at[pl.ds(r0, 1000), :],
                      acc_out_ref.at[hp, pl.ds(r0, 1000), :])


def _mm_kernel(x_ref, w_ref, o_ref):
  o_ref[...] = jnp.dot(x_ref[...], w_ref[...], preferred_element_type=_f32)


def _mm_bn_kernel(x_ref, w_ref, sums_ref, sq_ref, g_ref, b_ref, o_ref):
  mean = sums_ref[...] * (1.0 / N)
  var = sq_ref[...] * (1.0 / N) - mean * mean
  s = g_ref[...] * lax.rsqrt(var + 1e-5)
  t = b_ref[...] - mean * s
  o_ref[...] = jnp.dot(x_ref[...] * s + t, w_ref[...],
                       preferred_element_type=_f32)


def _ewm_kernel(ea_ref, wrep_ref, o_ref):
  colsum = jnp.sum(ea_ref[...], axis=0, keepdims=True)
  o_ref[...] = jnp.dot(colsum, wrep_ref[...],
                       preferred_element_type=_f32) * (1.0 / E)


def _gat_out_block(a0, a1, a2, a3, xl_ref, xr_ref, ewm_ref,
                   attr_ref, gt_ref, bias_ref):
  xl = xl_ref[...]
  m = xl + xr_ref[...] + ewm_ref[...]
  m = jnp.maximum(m, m * 0.2)
  malpha = m * attr_ref[...]
  al = jnp.dot(malpha, jnp.transpose(gt_ref[...]),
               preferred_element_type=_f32)
  al = jnp.exp(al)
  parts = [a0[...], a1[...], a2[...], a3[...]]
  dens = jnp.concatenate(
      [p[:, c:c + 1] for p in parts for c in (C, CP + C)], axis=1)
  den = dens + al + 1e-16
  alb = jnp.dot(al, gt_ref[...], preferred_element_type=_f32)
  denb = jnp.dot(den, gt_ref[...], preferred_element_type=_f32)
  acc = jnp.concatenate(parts, axis=1)
  return (acc + alb * xl) / denb + bias_ref[...]


def _post1_kernel(a0, a1, a2, a3, xl_ref, xr_ref, ewm_ref,
                  attr_ref, gt_ref, bias_ref, h_ref, sums_ref, sq_ref):
  i = pl.program_id(0)

  @pl.when(i == 0)
  def _():
    sums_ref[...] = jnp.zeros_like(sums_ref)
    sq_ref[...] = jnp.zeros_like(sq_ref)

  o = _gat_out_block(a0, a1, a2, a3, xl_ref, xr_ref, ewm_ref,
                     attr_ref, gt_ref, bias_ref)
  h = jnp.where(o > 0, o, jnp.exp(o) - 1.0)
  h_ref[...] = h
  sums_ref[...] += jnp.sum(h, axis=0, keepdims=True)
  sq_ref[...] += jnp.sum(h * h, axis=0, keepdims=True)


def _post2_kernel(a0, a1, a2, a3, xl_ref, xr_ref, ewm_ref,
                  attr_ref, gt_ref, bias_ref, batch_ref, g2_ref, b2_ref,
                  wfc_ref, bfc_ref, o_ref, gsum_ref, gcnt_ref):
  i = pl.program_id(0)
  nsteps = pl.num_programs(0)

  @pl.when(i == 0)
  def _():
    gsum_ref[...] = jnp.zeros_like(gsum_ref)
    gcnt_ref[...] = jnp.zeros_like(gcnt_ref)

  o = _gat_out_block(a0, a1, a2, a3, xl_ref, xr_ref, ewm_ref,
                     attr_ref, gt_ref, bias_ref)
  onehot = (batch_ref[...] ==
            lax.broadcasted_iota(_i32, (1, B), 1)).astype(_f32)
  gsum_ref[...] += lax.dot_general(onehot, o, (((0,), (0,)), ((), ())),
                                   preferred_element_type=_f32)
  gcnt_ref[...] += lax.dot_general(
      onehot, jnp.ones((onehot.shape[0], 128), _f32),
      (((0,), (0,)), ((), ())), preferred_element_type=_f32)

  @pl.when(i == nsteps - 1)
  def _():
    cnt = gcnt_ref[...][:, :1]
    g = gsum_ref[...] / jnp.maximum(cnt, 1.0)
    mean = jnp.mean(g, axis=0, keepdims=True)
    var = jnp.mean((g - mean) ** 2, axis=0, keepdims=True)
    gbn = (g - mean) * lax.rsqrt(var + 1e-5) * g2_ref[...] + b2_ref[...]
    logits = jnp.dot(gbn, wfc_ref[...], preferred_element_type=_f32)
    logits = logits + bfc_ref[...]
    mx = jnp.max(logits, axis=1, keepdims=True)
    z = logits - mx
    lse = jnp.log(jnp.sum(jnp.exp(z), axis=1, keepdims=True))
    o_ref[...] = (z - lse)[:, :NCLS]


def _mm(x, w):
  n, d = x.shape
  dout = w.shape[1]
  bn = 1000
  return pl.pallas_call(
      _mm_kernel,
      grid=(n // bn,),
      in_specs=[pl.BlockSpec((bn, d), lambda i: (i, 0)),
                pl.BlockSpec((d, dout), lambda i: (0, 0))],
      out_specs=pl.BlockSpec((bn, dout), lambda i: (i, 0)),
      out_shape=jax.ShapeDtypeStruct((n, dout), _f32),
  )(x, w)


def _mm_bn(x, w, sums, sq, gammap, betap):
  n, d = x.shape
  dout = w.shape[1]
  bn = 1000
  vec = lambda: pl.BlockSpec((1, d), lambda i: (0, 0))
  return pl.pallas_call(
      _mm_bn_kernel,
      grid=(n // bn,),
      in_specs=[pl.BlockSpec((bn, d), lambda i: (i, 0)),
                pl.BlockSpec((d, dout), lambda i: (0, 0)),
                vec(), vec(), vec(), vec()],
      out_specs=pl.BlockSpec((bn, dout), lambda i: (i, 0)),
      out_shape=jax.ShapeDtypeStruct((n, dout), _f32),
  )(x, w, sums, sq, gammap, betap)


def _ewm(ea_flat, wrep):
  return pl.pallas_call(
      _ewm_kernel,
      out_shape=jax.ShapeDtypeStruct((1, HCP), _f32),
  )(ea_flat, wrep)


def _sc_attn(srce, dste, ea_lin, xlp_rows, xrp_rows, wep_sc, attp):
  mesh = plsc.VectorSubcoreMesh(core_axis_name="c", subcore_axis_name="s",
                                num_cores=NCORE, num_subcores=NSUB)
  fn = pl.kernel(
      _attn_body,
      out_type=jax.ShapeDtypeStruct((H * E,), _f32),
      mesh=mesh,
      compiler_params=pltpu.CompilerParams(needs_layout_passes=False),
      scratch_types=[
          pltpu.VMEM((K,), _i32),
          pltpu.VMEM((K,), _i32),
          pltpu.VMEM((K * DE,), _f32),
          pltpu.VMEM((K,), _i32),
          pltpu.VMEM((K,), _i32),
          pltpu.VMEM((K, 2 * CP), _f32),
          pltpu.VMEM((K, 2 * CP), _f32),
          pltpu.VMEM((K,), _f32),
          pltpu.VMEM((K,), _f32),
          pltpu.VMEM((DE * H, CP), _f32),
          pltpu.VMEM((H, CP), _f32),
          pltpu.SemaphoreType.DMA,
          pltpu.SemaphoreType.DMA,
      ],
  )
  return fn(srce, dste, ea_lin, xlp_rows, xrp_rows, wep_sc, attp)


def _sc_aggr(srce, dste, alpha, xlp_rows):
  mesh = plsc.VectorSubcoreMesh(core_axis_name="c", subcore_axis_name="s",
                                num_cores=NCORE, num_subcores=NSUB)
  fn = pl.kernel(
      _aggr_body,
      out_type=jax.ShapeDtypeStruct((HP, N, 2 * CP), _f32),
      mesh=mesh,
      compiler_params=pltpu.CompilerParams(needs_layout_passes=False),
      scratch_types=[
          pltpu.VMEM((K,), _i32),
          pltpu.VMEM((K,), _i32),
          pltpu.VMEM((80,), _i32),
          pltpu.VMEM((80,), _i32),
          pltpu.VMEM((80, 2 * CP), _f32),
          pltpu.VMEM((K,), _f32),
          pltpu.VMEM((K,), _f32),
          pltpu.VMEM_SHARED((N, 2 * CP), _f32),
          pltpu.SemaphoreType.DMA,
      ],
  )
  return fn(srce, dste, alpha, xlp_rows)


def _post(kernel_fn, acc_parts, xlp, xrp, ewm, attr, gt_mat, biasp,
          extra_inputs, extra_specs, out_shapes, out_specs, scratch_shapes):
  bn = 1000
  part = lambda: pl.BlockSpec((bn, 2 * CP), lambda i: (i, 0))
  vec = lambda: pl.BlockSpec((1, HCP), lambda i: (0, 0))
  return pl.pallas_call(
      kernel_fn,
      grid=(N // bn,),
      in_specs=[part(), part(), part(), part(),
                pl.BlockSpec((bn, HCP), lambda i: (i, 0)),
                pl.BlockSpec((bn, HCP), lambda i: (i, 0)),
                vec(), vec(),
                pl.BlockSpec((H, HCP), lambda i: (0, 0)),
                vec()] + extra_specs,
      out_specs=out_specs,
      out_shape=out_shapes,
      scratch_shapes=scratch_shapes,
      compiler_params=pltpu.CompilerParams(
          dimension_semantics=("arbitrary",)),
  )(acc_parts[0], acc_parts[1], acc_parts[2], acc_parts[3],
    xlp, xrp, ewm, attr, gt_mat, biasp, *extra_inputs)


def kernel(x, edge_index, edge_attr, batch, Wl1, Wr1, We1, att1, b1,
           Wl2, Wr2, We2, att2, b2, gamma, beta, Wfc, bfc):
  def pad_cols(w):
    d = w.shape[0]
    return jnp.pad(w.reshape(d, H, C), ((0, 0), (0, 0), (0, CP - C))
                   ).reshape(d, HCP)

  def pad_rows(w):
    d = w.shape[1]
    return jnp.pad(w.reshape(H, C, d), ((0, 0), (0, CP - C), (0, 0))
                   ).reshape(HCP, d)

  def pad_vec(v):
    return jnp.pad(v.reshape(H, C), ((0, 0), (0, CP - C))).reshape(1, HCP)

  srce = edge_index[0].astype(_i32)
  dste = edge_index[1].astype(_i32)
  ea_lin = edge_attr.reshape(E * DE)
  wcat1 = jnp.concatenate([pad_cols(Wl1), pad_cols(Wr1)], axis=1)
  wcat2 = jnp.concatenate([pad_rows(pad_cols(Wl2)),
                           pad_rows(pad_cols(Wr2))], axis=1)
  wep1 = pad_cols(We1)
  wep2 = pad_cols(We2)
  wep1_sc = wep1.reshape(DE * H, CP)
  wep2_sc = wep2.reshape(DE * H, CP)
  wrep1 = jnp.tile(wep1, (1280 // DE, 1))
  wrep2 = jnp.tile(wep2, (1280 // DE, 1))
  attp1 = jnp.pad(att1, ((0, 0), (0, CP - C)))
  attp2 = jnp.pad(att2, ((0, 0), (0, CP - C)))
  attr1 = attp1.reshape(1, HCP)
  attr2 = attp2.reshape(1, HCP)
  b1p = pad_vec(b1)
  b2p = pad_vec(b2)
  gammap = pad_vec(gamma)
  betap = pad_vec(beta)
  wfcp = jnp.pad(pad_rows(Wfc), ((0, 0), (0, 128 - NCLS)))
  bfcp = jnp.pad(bfc, (0, 128 - NCLS), constant_values=-1e30).reshape(1, 128)
  gt_mat = (jnp.arange(HCP)[None, :] // CP ==
            jnp.arange(H)[:, None]).astype(_f32)
  ea_flat = edge_attr.reshape(E * DE // 1280, 1280)
  batch2d = batch.astype(_i32).reshape(N, 1)

  xlr1 = _mm(x, wcat1)
  xlp1 = xlr1[:, :HCP]
  xrp1 = xlr1[:, HCP:]
  xlp1r = xlp1.reshape(NR, 2 * CP)
  xrp1r = xrp1.reshape(NR, 2 * CP)
  ewm1 = _ewm(ea_flat, wrep1)
  alpha1 = _sc_attn(srce, dste, ea_lin, xlp1r, xrp1r, wep1_sc, attp1)
  acc1 = _sc_aggr(srce, dste, alpha1, xlp1r)
  acc1_parts = [acc1[i] for i in range(HP)]
  h1, sums1, sq1 = _post(
      _post1_kernel, acc1_parts, xlp1, xrp1, ewm1, attr1, gt_mat,
      b1p, [], [],
      [jax.ShapeDtypeStruct((N, HCP), _f32),
       jax.ShapeDtypeStruct((1, HCP), _f32),
       jax.ShapeDtypeStruct((1, HCP), _f32)],
      [pl.BlockSpec((1000, HCP), lambda i: (i, 0)),
       pl.BlockSpec((1, HCP), lambda i: (0, 0)),
       pl.BlockSpec((1, HCP), lambda i: (0, 0))],
      [])

  xlr2 = _mm_bn(h1, wcat2, sums1, sq1, gammap, betap)
  xlp2 = xlr2[:, :HCP]
  xrp2 = xlr2[:, HCP:]
  xlp2r = xlp2.reshape(NR, 2 * CP)
  xrp2r = xrp2.reshape(NR, 2 * CP)
  ewm2 = _ewm(ea_flat, wrep2)
  alpha2 = _sc_attn(srce, dste, ea_lin, xlp2r, xrp2r, wep2_sc, attp2)
  acc2 = _sc_aggr(srce, dste, alpha2, xlp2r)
  acc2_parts = [acc2[i] for i in range(HP)]

  out = _post(
      _post2_kernel, acc2_parts, xlp2, xrp2, ewm2, attr2, gt_mat,
      b2p,
      [batch2d, gammap, betap, wfcp, bfcp],
      [pl.BlockSpec((1000, 1), lambda i: (i, 0)),
       pl.BlockSpec((1, HCP), lambda i: (0, 0)),
       pl.BlockSpec((1, HCP), lambda i: (0, 0)),
       pl.BlockSpec((HCP, 128), lambda i: (0, 0)),
       pl.BlockSpec((1, 128), lambda i: (0, 0))],
      jax.ShapeDtypeStruct((B, NCLS), _f32),
      pl.BlockSpec((B, NCLS), lambda i: (0, 0)),
      [pltpu.VMEM((B, HCP), _f32), pltpu.VMEM((B, 128), _f32)])
  return out


# speedup vs baseline: 19.1423x; 1.3282x over previous
import jax
import jax.numpy as jnp
from jax import lax
from jax.experimental import pallas as pl
from jax.experimental.pallas import tpu as pltpu
from jax.experimental.pallas import tpu_sc as plsc

N = 10000
E = 320000
B = 32
DE = 5
H = 8
HP = 4
C = 56
CP = 64
HC = H * C
HCP = H * CP
NCLS = 18

NCORE = 2
NSUB = 16
NW = NCORE * NSUB

K = 400
EPW_A = E // NW
NCH_A = EPW_A // K
EPW_B = E // NSUB
NCH_B = EPW_B // K
NR = N * HP

_f32 = jnp.float32
_i32 = jnp.int32


def _attn_body(srce_ref, dste_ref, ea_ref, xlp_ref, xrp_ref, wep_ref, attp_ref,
               alpha_ref,
               src_v, dst_v, eav,
               idxs0, idxd0, idxs1, idxd1, xls0, xrd0, xls1, xrd1,
               alph_a, alph_b, wev, attv, sem, sem2):
  cid = lax.axis_index("c")
  sid = lax.axis_index("s")
  wid = sid * NCORE + cid
  KG = 80
  NGR = K // KG

  pltpu.sync_copy(wep_ref, wev)
  pltpu.sync_copy(attp_ref, attv)
  lane15 = lax.broadcasted_iota(_i32, (16,), 0) == 15

  idxs = [idxs0, idxs1]
  idxd = [idxd0, idxd1]
  xls = [xls0, xls1]
  xrd = [xrd0, xrd1]

  def mk_idx(g, hp, pn):
    @plsc.parallel_loop(0, KG // 16, 1)
    def _(t):
      slo = pl.ds(t * 16, 16)
      sli = pl.ds(g * KG + t * 16, 16)
      idxs[pn][slo] = src_v[sli] * HP + hp
      idxd[pn][slo] = dst_v[sli] * HP + hp

  def issue(pn):
    d1 = pltpu.async_copy(xlp_ref.at[idxs[pn]], xls[pn], sem)
    d2 = pltpu.async_copy(xrp_ref.at[idxd[pn]], xrd[pn], sem2)
    return (d1, d2)

  def chunk_body(i, _):
    base = wid * EPW_A + i * K
    pltpu.sync_copy(srce_ref.at[pl.ds(base, K)], src_v)
    pltpu.sync_copy(dste_ref.at[pl.ds(base, K)], dst_v)
    pltpu.sync_copy(ea_ref.at[pl.ds(base * DE, K * DE)], eav)

    mk_idx(0, 0, 0)
    desc = [None, None]
    desc[0] = issue(0)
    for hp in range(HP):
      wq = [[wev[j * H + hp * 2 + q // 4, pl.ds((q % 4) * 16, 16)]
             for q in range(8)] for j in range(DE)]
      aq = [attv[hp * 2 + q // 4, pl.ds((q % 4) * 16, 16)] for q in range(8)]
      for g in range(NGR):
        u = hp * NGR + g
        pn = u % 2
        desc[pn][0].wait()
        desc[pn][1].wait()
        if u < HP * NGR - 1:
          nhp, ng = (hp, g + 1) if g < NGR - 1 else (hp + 1, 0)
          mk_idx(ng, nhp, 1 - pn)
          desc[1 - pn] = issue(1 - pn)

        @plsc.parallel_loop(0, KG, 1, unroll=1)
        def _edge(e):
          esplat = jnp.full((16,), g * KG + e, _i32)
          ebase = jnp.full((16,), (g * KG) * DE, _i32) + e * DE
          ea_j = [plsc.load_gather(eav, [ebase + j]) for j in range(DE)]
          acc0 = None
          acc1 = None
          for q in range(8):
            sl = pl.ds(q * 16, 16)
            m = xls[pn][e, sl] + xrd[pn][e, sl]
            for j in range(DE):
              m = m + ea_j[j] * wq[j][q]
            m = jnp.maximum(m, m * 0.2)
            t = m * aq[q]
            if q < 4:
              acc0 = t if acc0 is None else acc0 + t
            else:
              acc1 = t if acc1 is None else acc1 + t
          plsc.store_scatter(alph_a, [esplat], plsc.cumsum(acc0), mask=lane15)
          plsc.store_scatter(alph_b, [esplat], plsc.cumsum(acc1), mask=lane15)

      @plsc.parallel_loop(0, K // 16, 1, unroll=2)
      def _expo(t):
        sl = pl.ds(t * 16, 16)
        alph_a[sl] = jnp.exp(alph_a[sl])
        alph_b[sl] = jnp.exp(alph_b[sl])

      pltpu.sync_copy(alph_a, alpha_ref.at[pl.ds((hp * 2) * E + base, K)])
      pltpu.sync_copy(alph_b, alpha_ref.at[pl.ds((hp * 2 + 1) * E + base, K)])
    return 0

  lax.fori_loop(0, NCH_A, chunk_body, 0)


def _aggr_body(srce_ref, dste_ref, alpha_ref, xlp_ref,
               acc_out_ref,
               src_v, dst_v,
               idx0, idx1, dst80_0, dst80_1, rows0, rows1,
               alph_a, alph_b, acc_sh, sem, sem3):
  cid = lax.axis_index("c")
  sid = lax.axis_index("s")
  lane8 = lax.broadcasted_iota(_i32, (16,), 0) == 8
  ones16 = jnp.ones((16,), _f32)
  KG = 80
  KB = 800
  NCHB = EPW_B // KB
  NGR = KB // KG

  idx = [idx0, idx1]
  dst80 = [dst80_0, dst80_1]
  rows = [rows0, rows1]

  def mk_idx(g, hp, pn):
    @plsc.parallel_loop(0, KG // 16, 1)
    def _(t):
      slo = pl.ds(t * 16, 16)
      sli = pl.ds(g * KG + t * 16, 16)
      idx[pn][slo] = src_v[sli] * HP + hp
      dst80[pn][slo] = dst_v[sli]

  for p in range(2):
    hp = cid * 2 + p

    @plsc.parallel_loop(0, KG, 1, unroll=4)
    def _zero_rows(e):
      for q in range(8):
        rows0[e, pl.ds(q * 16, 16)] = jnp.zeros((16,), _f32)

    @pl.when(sid < 10)
    def _():
      r0 = sid * 1000
      for z in range(12):
        pltpu.sync_copy(rows0, acc_sh.at[pl.ds(r0 + z * KG, KG), :])
      pltpu.sync_copy(rows0.at[pl.ds(0, 40), :],
                      acc_sh.at[pl.ds(r0 + 12 * KG, 40), :])
    plsc.subcore_barrier()

    def chunk_body(i, _):
      base = sid * EPW_B + i * KB
      pltpu.sync_copy(srce_ref.at[pl.ds(base, KB)], src_v)
      pltpu.sync_copy(dste_ref.at[pl.ds(base, KB)], dst_v)
      pltpu.sync_copy(alpha_ref.at[pl.ds((hp * 2) * E + base, KB)], alph_a)
      pltpu.sync_copy(alpha_ref.at[pl.ds((hp * 2 + 1) * E + base, KB)],
                      alph_b)

      mk_idx(0, hp, 0)
      gd = [None, None]
      sd = [None, None]
      gd[0] = pltpu.async_copy(xlp_ref.at[idx[0]], rows[0], sem)
      for g in range(NGR):
        pn = g % 2
        if sd[1 - pn] is not None:
          sd[1 - pn].wait()
          sd[1 - pn] = None
        gd[pn].wait()
        if g < NGR - 1:
          mk_idx(g + 1, hp, 1 - pn)
          gd[1 - pn] = pltpu.async_copy(xlp_ref.at[idx[1 - pn]],
                                        rows[1 - pn], sem)

        @plsc.parallel_loop(0, KG, 1, unroll=2)
        def _scale(e):
          esplat = jnp.full((16,), g * KG + e, _i32)
          a0 = plsc.load_gather(alph_a, [esplat])
          a1 = plsc.load_gather(alph_b, [esplat])
          for q in range(8):
            sl = pl.ds(q * 16, 16)
            r = rows[pn][e, sl]
            if q == 3 or q == 7:
              r = jnp.where(lane8, ones16, r)
            rows[pn][e, sl] = r * (a0 if q < 4 else a1)

        sd[pn] = pltpu.async_copy(rows[pn], acc_sh.at[dst80[pn]], sem3,
                                  add=True)
      for pn in range(2):
        if sd[pn] is not None:
          sd[pn].wait()
      return 0

    lax.fori_loop(0, NCHB, chunk_body, 0)
    plsc.subcore_barrier()

    @pl.when(sid < 10)
    def _():
      r0 = sid * 1000
      pltpu.sync_copy(acc_sh.at[pl.ds(r0, 1000), :],
                      acc_out_ref.at[hp, pl.ds(r0, 1000), :])


def _mm_kernel(x_ref, w_ref, o_ref):
  o_ref[...] = jnp.dot(x_ref[...], w_ref[...], preferred_element_type=_f32)


def _mm_bn_kernel(x_ref, w_ref, sums_ref, sq_ref, g_ref, b_ref, o_ref):
  mean = sums_ref[...] * (1.0 / N)
  var = sq_ref[...] * (1.0 / N) - mean * mean
  s = g_ref[...] * lax.rsqrt(var + 1e-5)
  t = b_ref[...] - mean * s
  o_ref[...] = jnp.dot(x_ref[...] * s + t, w_ref[...],
                       preferred_element_type=_f32)


def _ewm_kernel(ea_ref, wrep_ref, o_ref):
  colsum = jnp.sum(ea_ref[...], axis=0, keepdims=True)
  o_ref[...] = jnp.dot(colsum, wrep_ref[...],
                       preferred_element_type=_f32) * (1.0 / E)


def _gat_out_block(a0, a1, a2, a3, xl_ref, xr_ref, ewm_ref,
                   attr_ref, gt_ref, bias_ref):
  xl = xl_ref[...]
  m = xl + xr_ref[...] + ewm_ref[...]
  m = jnp.maximum(m, m * 0.2)
  malpha = m * attr_ref[...]
  al = jnp.dot(malpha, jnp.transpose(gt_ref[...]),
               preferred_element_type=_f32)
  al = jnp.exp(al)
  parts = [a0[...], a1[...], a2[...], a3[...]]
  dens = jnp.concatenate(
      [p[:, c:c + 1] for p in parts for c in (C, CP + C)], axis=1)
  den = dens + al + 1e-16
  alb = jnp.dot(al, gt_ref[...], preferred_element_type=_f32)
  denb = jnp.dot(den, gt_ref[...], preferred_element_type=_f32)
  acc = jnp.concatenate(parts, axis=1)
  return (acc + alb * xl) / denb + bias_ref[...]


def _post1_kernel(a0, a1, a2, a3, xl_ref, xr_ref, ewm_ref,
                  attr_ref, gt_ref, bias_ref, h_ref, sums_ref, sq_ref):
  i = pl.program_id(0)

  @pl.when(i == 0)
  def _():
    sums_ref[...] = jnp.zeros_like(sums_ref)
    sq_ref[...] = jnp.zeros_like(sq_ref)

  o = _gat_out_block(a0, a1, a2, a3, xl_ref, xr_ref, ewm_ref,
                     attr_ref, gt_ref, bias_ref)
  h = jnp.where(o > 0, o, jnp.exp(o) - 1.0)
  h_ref[...] = h
  sums_ref[...] += jnp.sum(h, axis=0, keepdims=True)
  sq_ref[...] += jnp.sum(h * h, axis=0, keepdims=True)


def _post2_kernel(a0, a1, a2, a3, xl_ref, xr_ref, ewm_ref,
                  attr_ref, gt_ref, bias_ref, batch_ref, g2_ref, b2_ref,
                  wfc_ref, bfc_ref, o_ref, gsum_ref, gcnt_ref):
  i = pl.program_id(0)
  nsteps = pl.num_programs(0)

  @pl.when(i == 0)
  def _():
    gsum_ref[...] = jnp.zeros_like(gsum_ref)
    gcnt_ref[...] = jnp.zeros_like(gcnt_ref)

  o = _gat_out_block(a0, a1, a2, a3, xl_ref, xr_ref, ewm_ref,
                     attr_ref, gt_ref, bias_ref)
  onehot = (batch_ref[...] ==
            lax.broadcasted_iota(_i32, (1, B), 1)).astype(_f32)
  gsum_ref[...] += lax.dot_general(onehot, o, (((0,), (0,)), ((), ())),
                                   preferred_element_type=_f32)
  gcnt_ref[...] += lax.dot_general(
      onehot, jnp.ones((onehot.shape[0], 128), _f32),
      (((0,), (0,)), ((), ())), preferred_element_type=_f32)

  @pl.when(i == nsteps - 1)
  def _():
    cnt = gcnt_ref[...][:, :1]
    g = gsum_ref[...] / jnp.maximum(cnt, 1.0)
    mean = jnp.mean(g, axis=0, keepdims=True)
    var = jnp.mean((g - mean) ** 2, axis=0, keepdims=True)
    gbn = (g - mean) * lax.rsqrt(var + 1e-5) * g2_ref[...] + b2_ref[...]
    logits = jnp.dot(gbn, wfc_ref[...], preferred_element_type=_f32)
    logits = logits + bfc_ref[...]
    mx = jnp.max(logits, axis=1, keepdims=True)
    z = logits - mx
    lse = jnp.log(jnp.sum(jnp.exp(z), axis=1, keepdims=True))
    o_ref[...] = (z - lse)[:, :NCLS]


def _mm(x, w):
  n, d = x.shape
  dout = w.shape[1]
  bn = 1000
  return pl.pallas_call(
      _mm_kernel,
      grid=(n // bn,),
      in_specs=[pl.BlockSpec((bn, d), lambda i: (i, 0)),
                pl.BlockSpec((d, dout), lambda i: (0, 0))],
      out_specs=pl.BlockSpec((bn, dout), lambda i: (i, 0)),
      out_shape=jax.ShapeDtypeStruct((n, dout), _f32),
  )(x, w)


def _mm_bn(x, w, sums, sq, gammap, betap):
  n, d = x.shape
  dout = w.shape[1]
  bn = 1000
  vec = lambda: pl.BlockSpec((1, d), lambda i: (0, 0))
  return pl.pallas_call(
      _mm_bn_kernel,
      grid=(n // bn,),
      in_specs=[pl.BlockSpec((bn, d), lambda i: (i, 0)),
                pl.BlockSpec((d, dout), lambda i: (0, 0)),
                vec(), vec(), vec(), vec()],
      out_specs=pl.BlockSpec((bn, dout), lambda i: (i, 0)),
      out_shape=jax.ShapeDtypeStruct((n, dout), _f32),
  )(x, w, sums, sq, gammap, betap)


def _ewm(ea_flat, wrep):
  return pl.pallas_call(
      _ewm_kernel,
      out_shape=jax.ShapeDtypeStruct((1, HCP), _f32),
  )(ea_flat, wrep)


def _sc_attn(srce, dste, ea_lin, xlp_rows, xrp_rows, wep_sc, attp):
  mesh = plsc.VectorSubcoreMesh(core_axis_name="c", subcore_axis_name="s",
                                num_cores=NCORE, num_subcores=NSUB)
  fn = pl.kernel(
      _attn_body,
      out_type=jax.ShapeDtypeStruct((H * E,), _f32),
      mesh=mesh,
      compiler_params=pltpu.CompilerParams(needs_layout_passes=False),
      scratch_types=[
          pltpu.VMEM((K,), _i32),
          pltpu.VMEM((K,), _i32),
          pltpu.VMEM((K * DE,), _f32),
          pltpu.VMEM((80,), _i32),
          pltpu.VMEM((80,), _i32),
          pltpu.VMEM((80,), _i32),
          pltpu.VMEM((80,), _i32),
          pltpu.VMEM((80, 2 * CP), _f32),
          pltpu.VMEM((80, 2 * CP), _f32),
          pltpu.VMEM((80, 2 * CP), _f32),
          pltpu.VMEM((80, 2 * CP), _f32),
          pltpu.VMEM((K,), _f32),
          pltpu.VMEM((K,), _f32),
          pltpu.VMEM((DE * H, CP), _f32),
          pltpu.VMEM((H, CP), _f32),
          pltpu.SemaphoreType.DMA,
          pltpu.SemaphoreType.DMA,
      ],
  )
  return fn(srce, dste, ea_lin, xlp_rows, xrp_rows, wep_sc, attp)


def _sc_aggr(srce, dste, alpha, xlp_rows):
  mesh = plsc.VectorSubcoreMesh(core_axis_name="c", subcore_axis_name="s",
                                num_cores=NCORE, num_subcores=NSUB)
  fn = pl.kernel(
      _aggr_body,
      out_type=jax.ShapeDtypeStruct((HP, N, 2 * CP), _f32),
      mesh=mesh,
      compiler_params=pltpu.CompilerParams(needs_layout_passes=False),
      scratch_types=[
          pltpu.VMEM((800,), _i32),
          pltpu.VMEM((800,), _i32),
          pltpu.VMEM((80,), _i32),
          pltpu.VMEM((80,), _i32),
          pltpu.VMEM((80,), _i32),
          pltpu.VMEM((80,), _i32),
          pltpu.VMEM((80, 2 * CP), _f32),
          pltpu.VMEM((80, 2 * CP), _f32),
          pltpu.VMEM((800,), _f32),
          pltpu.VMEM((800,), _f32),
          pltpu.VMEM_SHARED((N, 2 * CP), _f32),
          pltpu.SemaphoreType.DMA,
          pltpu.SemaphoreType.DMA,
      ],
  )
  return fn(srce, dste, alpha, xlp_rows)


def _post(kernel_fn, acc_parts, xlp, xrp, ewm, attr, gt_mat, biasp,
          extra_inputs, extra_specs, out_shapes, out_specs, scratch_shapes):
  bn = 1000
  part = lambda: pl.BlockSpec((bn, 2 * CP), lambda i: (i, 0))
  vec = lambda: pl.BlockSpec((1, HCP), lambda i: (0, 0))
  return pl.pallas_call(
      kernel_fn,
      grid=(N // bn,),
      in_specs=[part(), part(), part(), part(),
                pl.BlockSpec((bn, HCP), lambda i: (i, 0)),
                pl.BlockSpec((bn, HCP), lambda i: (i, 0)),
                vec(), vec(),
                pl.BlockSpec((H, HCP), lambda i: (0, 0)),
                vec()] + extra_specs,
      out_specs=out_specs,
      out_shape=out_shapes,
      scratch_shapes=scratch_shapes,
      compiler_params=pltpu.CompilerParams(
          dimension_semantics=("arbitrary",)),
  )(acc_parts[0], acc_parts[1], acc_parts[2], acc_parts[3],
    xlp, xrp, ewm, attr, gt_mat, biasp, *extra_inputs)


def kernel(x, edge_index, edge_attr, batch, Wl1, Wr1, We1, att1, b1,
           Wl2, Wr2, We2, att2, b2, gamma, beta, Wfc, bfc):
  def pad_cols(w):
    d = w.shape[0]
    return jnp.pad(w.reshape(d, H, C), ((0, 0), (0, 0), (0, CP - C))
                   ).reshape(d, HCP)

  def pad_rows(w):
    d = w.shape[1]
    return jnp.pad(w.reshape(H, C, d), ((0, 0), (0, CP - C), (0, 0))
                   ).reshape(HCP, d)

  def pad_vec(v):
    return jnp.pad(v.reshape(H, C), ((0, 0), (0, CP - C))).reshape(1, HCP)

  srce = edge_index[0].astype(_i32)
  dste = edge_index[1].astype(_i32)
  ea_lin = edge_attr.reshape(E * DE)
  wcat1 = jnp.concatenate([pad_cols(Wl1), pad_cols(Wr1)], axis=1)
  wcat2 = jnp.concatenate([pad_rows(pad_cols(Wl2)),
                           pad_rows(pad_cols(Wr2))], axis=1)
  wep1 = pad_cols(We1)
  wep2 = pad_cols(We2)
  wep1_sc = wep1.reshape(DE * H, CP)
  wep2_sc = wep2.reshape(DE * H, CP)
  wrep1 = jnp.tile(wep1, (1280 // DE, 1))
  wrep2 = jnp.tile(wep2, (1280 // DE, 1))
  attp1 = jnp.pad(att1, ((0, 0), (0, CP - C)))
  attp2 = jnp.pad(att2, ((0, 0), (0, CP - C)))
  attr1 = attp1.reshape(1, HCP)
  attr2 = attp2.reshape(1, HCP)
  b1p = pad_vec(b1)
  b2p = pad_vec(b2)
  gammap = pad_vec(gamma)
  betap = pad_vec(beta)
  wfcp = jnp.pad(pad_rows(Wfc), ((0, 0), (0, 128 - NCLS)))
  bfcp = jnp.pad(bfc, (0, 128 - NCLS), constant_values=-1e30).reshape(1, 128)
  gt_mat = (jnp.arange(HCP)[None, :] // CP ==
            jnp.arange(H)[:, None]).astype(_f32)
  ea_flat = edge_attr.reshape(E * DE // 1280, 1280)
  batch2d = batch.astype(_i32).reshape(N, 1)

  xlr1 = _mm(x, wcat1)
  xlp1 = xlr1[:, :HCP]
  xrp1 = xlr1[:, HCP:]
  xlp1r = xlp1.reshape(NR, 2 * CP)
  xrp1r = xrp1.reshape(NR, 2 * CP)
  ewm1 = _ewm(ea_flat, wrep1)
  alpha1 = _sc_attn(srce, dste, ea_lin, xlp1r, xrp1r, wep1_sc, attp1)
  acc1 = _sc_aggr(srce, dste, alpha1, xlp1r)
  acc1_parts = [acc1[i] for i in range(HP)]
  h1, sums1, sq1 = _post(
      _post1_kernel, acc1_parts, xlp1, xrp1, ewm1, attr1, gt_mat,
      b1p, [], [],
      [jax.ShapeDtypeStruct((N, HCP), _f32),
       jax.ShapeDtypeStruct((1, HCP), _f32),
       jax.ShapeDtypeStruct((1, HCP), _f32)],
      [pl.BlockSpec((1000, HCP), lambda i: (i, 0)),
       pl.BlockSpec((1, HCP), lambda i: (0, 0)),
       pl.BlockSpec((1, HCP), lambda i: (0, 0))],
      [])

  xlr2 = _mm_bn(h1, wcat2, sums1, sq1, gammap, betap)
  xlp2 = xlr2[:, :HCP]
  xrp2 = xlr2[:, HCP:]
  xlp2r = xlp2.reshape(NR, 2 * CP)
  xrp2r = xrp2.reshape(NR, 2 * CP)
  ewm2 = _ewm(ea_flat, wrep2)
  alpha2 = _sc_attn(srce, dste, ea_lin, xlp2r, xrp2r, wep2_sc, attp2)
  acc2 = _sc_aggr(srce, dste, alpha2, xlp2r)
  acc2_parts = [acc2[i] for i in range(HP)]

  out = _post(
      _post2_kernel, acc2_parts, xlp2, xrp2, ewm2, attr2, gt_mat,
      b2p,
      [batch2d, gammap, betap, wfcp, bfcp],
      [pl.BlockSpec((1000, 1), lambda i: (i, 0)),
       pl.BlockSpec((1, HCP), lambda i: (0, 0)),
       pl.BlockSpec((1, HCP), lambda i: (0, 0)),
       pl.BlockSpec((HCP, 128), lambda i: (0, 0)),
       pl.BlockSpec((1, 128), lambda i: (0, 0))],
      jax.ShapeDtypeStruct((B, NCLS), _f32),
      pl.BlockSpec((B, NCLS), lambda i: (0, 0)),
      [pltpu.VMEM((B, HCP), _f32), pltpu.VMEM((B, 128), _f32)])
  return out
